# Initial kernel scaffold; baseline (speedup 1.0000x reference)
import jax
import jax.numpy as jnp
from jax.experimental import pallas as pl


def kernel(x, edge_index, batch, W1, b1, W2, b2, W3, b3, lin_w, lin_b):
    raise NotImplementedError("write your pallas kernel here")


# speedup vs baseline: 9.0148x; 9.0148x over previous
import functools

import jax
import jax.numpy as jnp
from jax import lax
from jax.experimental import pallas as pl
from jax.experimental.pallas import tpu as pltpu
from jax.experimental.pallas import tpu_sc as plsc

NC = 2
NS = 16
NW = NC * NS


@functools.lru_cache(maxsize=None)
def _make_deg(N, E, CH, K, F):
    RPT = N // NS
    mesh = plsc.VectorSubcoreMesh(core_axis_name="c", subcore_axis_name="s")

    @functools.partial(
        pl.kernel,
        out_type=jax.ShapeDtypeStruct((NC, N, F), jnp.float32),
        mesh=mesh,
        scratch_types=[
            pltpu.VMEM((CH, K), jnp.int32),
            pltpu.VMEM((K, F), jnp.float32),
            pltpu.VMEM_SHARED((N, F), jnp.float32),
        ],
    )
    def deg_kernel(rows_hbm, ones_hbm, zeros_hbm, out_hbm, ridx, ones_v, acc):
        c = lax.axis_index("c")
        s = lax.axis_index("s")
        w = c * NS + s
        pltpu.sync_copy(rows_hbm.at[w], ridx)
        pltpu.sync_copy(ones_hbm, ones_v)
        pltpu.sync_copy(zeros_hbm, acc.at[pl.ds(s * RPT, RPT)])
        plsc.subcore_barrier()

        def body(j, carry):
            pltpu.sync_copy(ones_v, acc.at[ridx.at[j]], add=True)
            return carry

        lax.fori_loop(0, CH, body, 0)
        plsc.subcore_barrier()
        pltpu.sync_copy(acc.at[pl.ds(s * RPT, RPT)],
                        out_hbm.at[c, pl.ds(s * RPT, RPT)])

    return deg_kernel


@functools.lru_cache(maxsize=None)
def _make_prop(N, E, CH, K, F):
    RPT = N // NS
    mesh = plsc.VectorSubcoreMesh(core_axis_name="c", subcore_axis_name="s")

    @functools.partial(
        pl.kernel,
        out_type=jax.ShapeDtypeStruct((NC, N, F), jnp.float32),
        mesh=mesh,
        scratch_types=[
            pltpu.VMEM((CH, K), jnp.int32),
            pltpu.VMEM((CH, K), jnp.int32),
            pltpu.VMEM((K, F), jnp.float32),
            pltpu.VMEM_SHARED((N, F), jnp.float32),
            pltpu.SemaphoreType.DMA,
        ],
    )
    def prop_kernel(rows_hbm, cols_hbm, table_hbm, zeros_hbm, out_hbm,
                    ridx, cidx, buf, acc, gsem):
        c = lax.axis_index("c")
        s = lax.axis_index("s")
        w = c * NS + s
        pltpu.sync_copy(rows_hbm.at[w], ridx)
        pltpu.sync_copy(cols_hbm.at[w], cidx)
        pltpu.sync_copy(zeros_hbm, acc.at[pl.ds(s * RPT, RPT)])
        plsc.subcore_barrier()

        def body(j, carry):
            pltpu.async_copy(table_hbm.at[ridx.at[j]], buf, gsem).wait()
            pltpu.sync_copy(buf, acc.at[cidx.at[j]], add=True)
            return carry

        lax.fori_loop(0, CH, body, 0)
        plsc.subcore_barrier()
        pltpu.sync_copy(acc.at[pl.ds(s * RPT, RPT)],
                        out_hbm.at[c, pl.ds(s * RPT, RPT)])

    return prop_kernel


def _tc_prep(d0, d1, x, R):
    N, F = x.shape

    def body(d0_ref, d1_ref, x_ref, dinv_ref, g0_ref):
        deg = d0_ref[:, 0:1] + d1_ref[:, 0:1]
        dinv = jnp.where(deg > 0.0, lax.rsqrt(jnp.maximum(deg, 1e-30)), 0.0)
        dinv_ref[...] = dinv
        g0_ref[...] = dinv * x_ref[...]

    grid = N // R
    return pl.pallas_call(
        body,
        grid=(grid,),
        in_specs=[
            pl.BlockSpec((R, d0.shape[1]), lambda i: (i, 0)),
            pl.BlockSpec((R, d1.shape[1]), lambda i: (i, 0)),
            pl.BlockSpec((R, F), lambda i: (i, 0)),
        ],
        out_specs=[
            pl.BlockSpec((R, 1), lambda i: (i, 0)),
            pl.BlockSpec((R, F), lambda i: (i, 0)),
        ],
        out_shape=[
            jax.ShapeDtypeStruct((N, 1), jnp.float32),
            jax.ShapeDtypeStruct((N, F), jnp.float32),
        ],
    )(d0, d1, x)


def _tc_mid(p0, p1, dinv, R):
    N, F = p0.shape

    def body(p0_ref, p1_ref, dinv_ref, tx1_ref, g1_ref):
        dv = dinv_ref[...]
        tx1 = -dv * (p0_ref[...] + p1_ref[...])
        tx1_ref[...] = tx1
        g1_ref[...] = dv * tx1

    grid = N // R
    return pl.pallas_call(
        body,
        grid=(grid,),
        in_specs=[
            pl.BlockSpec((R, F), lambda i: (i, 0)),
            pl.BlockSpec((R, F), lambda i: (i, 0)),
            pl.BlockSpec((R, 1), lambda i: (i, 0)),
        ],
        out_specs=[
            pl.BlockSpec((R, F), lambda i: (i, 0)),
            pl.BlockSpec((R, F), lambda i: (i, 0)),
        ],
        out_shape=[
            jax.ShapeDtypeStruct((N, F), jnp.float32),
            jax.ShapeDtypeStruct((N, F), jnp.float32),
        ],
    )(p0, p1, dinv)


def _tc_layer(q0, q1, dinv, tx0, tx1, Wc, b, R, emit_next):
    N, F = tx0.shape
    H = Wc.shape[2]

    def body(q0_ref, q1_ref, dinv_ref, tx0_ref, tx1_ref, w_ref, b_ref, *outs):
        dv = dinv_ref[...]
        tx0v = tx0_ref[...]
        tx2 = -2.0 * dv * (q0_ref[...] + q1_ref[...]) - tx0v
        out = (jnp.dot(tx0v, w_ref[0], preferred_element_type=jnp.float32)
               + jnp.dot(tx1_ref[...], w_ref[1], preferred_element_type=jnp.float32)
               + jnp.dot(tx2, w_ref[2], preferred_element_type=jnp.float32)
               + b_ref[...])
        outs[0][...] = out
        if emit_next:
            hn = jnp.maximum(out, 0.0)
            outs[1][...] = hn
            outs[2][...] = dv * hn

    grid = N // R
    n_out = 3 if emit_next else 1
    return pl.pallas_call(
        body,
        grid=(grid,),
        in_specs=[
            pl.BlockSpec((R, F), lambda i: (i, 0)),
            pl.BlockSpec((R, F), lambda i: (i, 0)),
            pl.BlockSpec((R, 1), lambda i: (i, 0)),
            pl.BlockSpec((R, F), lambda i: (i, 0)),
            pl.BlockSpec((R, F), lambda i: (i, 0)),
            pl.BlockSpec((3, F, H), lambda i: (0, 0, 0)),
            pl.BlockSpec((1, H), lambda i: (0, 0)),
        ],
        out_specs=[pl.BlockSpec((R, H), lambda i: (i, 0))] * n_out,
        out_shape=[jax.ShapeDtypeStruct((N, H), jnp.float32)] * n_out,
    )(q0, q1, dinv, tx0, tx1, Wc, b)


def _tc_pool(h, batch_f, lin_w, lin_b, C):
    N, H = h.shape
    G = 64
    OUT = lin_w.shape[0]
    grid = N // C

    def body(h_ref, b_ref, w_ref, lb_ref, pooled_ref, out_ref, sums, cnt):
        i = pl.program_id(0)
        gids = lax.broadcasted_iota(jnp.int32, (G, C), 0).astype(jnp.float32)
        oh = jnp.where(gids == b_ref[0], 1.0, 0.0)
        psum = jnp.dot(oh, h_ref[...], preferred_element_type=jnp.float32)
        pcnt = jnp.sum(oh, axis=1, keepdims=True)

        @pl.when(i == 0)
        def _():
            sums[...] = psum
            cnt[...] = pcnt

        @pl.when(i > 0)
        def _():
            sums[...] = sums[...] + psum
            cnt[...] = cnt[...] + pcnt

        @pl.when(i == grid - 1)
        def _():
            pooled = sums[...] / jnp.maximum(cnt[...], 1.0)
            pooled_ref[...] = pooled
            out_ref[...] = lax.dot_general(
                pooled, w_ref[...], (((1,), (1,)), ((), ())),
                preferred_element_type=jnp.float32) + lb_ref[...]

    return pl.pallas_call(
        body,
        grid=(grid,),
        in_specs=[
            pl.BlockSpec((C, H), lambda i: (i, 0)),
            pl.BlockSpec((1, 1, C), lambda i: (i, 0, 0)),
            pl.BlockSpec((OUT, H), lambda i: (0, 0)),
            pl.BlockSpec((1, OUT), lambda i: (0, 0)),
        ],
        out_specs=[
            pl.BlockSpec((G, H), lambda i: (0, 0)),
            pl.BlockSpec((G, OUT), lambda i: (0, 0)),
        ],
        out_shape=[
            jax.ShapeDtypeStruct((G, H), jnp.float32),
            jax.ShapeDtypeStruct((G, OUT), jnp.float32),
        ],
        scratch_shapes=[
            pltpu.VMEM((G, H), jnp.float32),
            pltpu.VMEM((G, 1), jnp.float32),
        ],
    )(h, batch_f, lin_w, lin_b)


def kernel(x, edge_index, batch, W1, b1, W2, b2, W3, b3, lin_w, lin_b):
    N, D = x.shape
    E = edge_index.shape[1]
    H = W1.shape[1]
    K = 80
    CH = E // (NW * K)
    R = 2000
    DF = 128
    NP = ((N + 8 * NS - 1) // (8 * NS)) * (8 * NS)
    RPT = NP // NS

    rows3 = edge_index[0].reshape(NW, CH, K)
    cols3 = edge_index[1].reshape(NW, CH, K)
    zeros_h = jnp.zeros((RPT, H), jnp.float32)
    zeros8_h = jnp.zeros((RPT, DF), jnp.float32)
    ones_h = jnp.ones((K, DF), jnp.float32)

    deg_k = _make_deg(NP, E, CH, K, DF)
    prop_k = _make_prop(NP, E, CH, K, H)

    dpart = deg_k(rows3, ones_h, zeros8_h)
    dinv, g = _tc_prep(dpart[0, :N], dpart[1, :N], x, R)

    xs = []
    h = x
    for li, (W, b) in enumerate(((W1, b1), (W2, b2), (W3, b3))):
        Wc = jnp.transpose(W, (0, 2, 1))
        p = prop_k(rows3, cols3, g, zeros_h)
        tx1, g1 = _tc_mid(p[0, :N], p[1, :N], dinv, R)
        q = prop_k(rows3, cols3, g1, zeros_h)
        last = li == 2
        outs = _tc_layer(q[0, :N], q[1, :N], dinv, h, tx1, Wc,
                         b.reshape(1, H), R, emit_next=not last)
        xs.append(outs[0])
        if not last:
            h, g = outs[1], outs[2]

    h3 = xs[2]
    pooled, out = _tc_pool(h3, batch.astype(jnp.float32).reshape(N // 2000, 1, 2000),
                           lin_w, lin_b.reshape(1, lin_w.shape[0]), 2000)
    return (out, xs[0], xs[1], h3, pooled)
